# baseline (speedup 1.0000x reference)
import functools

import jax
import jax.numpy as jnp
from jax import lax
from jax.experimental import pallas as pl
from jax.experimental.pallas import tpu as pltpu
from jax.experimental.pallas import tpu_sc as plsc

B = 4096
L = 196
D = 64
MAX_T = 160
TBL = 168
BC = 128
NB = B // BC

SC_NC = 2
SC_NW = 32
BPW = B // SC_NW


def _sc_gather_body(ts_hbm, table_hbm, out_hbm, idx_v, rows_v, sem):
    wid = lax.axis_index("s") * SC_NC + lax.axis_index("c")
    base = wid * BPW
    pltpu.sync_copy(ts_hbm.at[pl.ds(base, BPW)], idx_v)
    pltpu.async_copy(table_hbm.at[idx_v], rows_v, sem).wait()
    pltpu.sync_copy(rows_v, out_hbm.at[pl.ds(base, BPW)])


_sc_gather = functools.partial(
    pl.kernel,
    out_type=jax.ShapeDtypeStruct((B, 128), jnp.float32),
    mesh=plsc.VectorSubcoreMesh(core_axis_name="c", subcore_axis_name="s"),
    scratch_types=[
        pltpu.VMEM((BPW,), jnp.int32),
        pltpu.VMEM((BPW, 128), jnp.float32),
        pltpu.SemaphoreType.DMA,
    ],
)(_sc_gather_body)


def _body(temb_ref, speed_ref, fcw_ref, fcb_ref, patch_ref, out_ref):
    t = pl.program_id(0)

    @pl.when(t == 0)
    def _dense():
        x = patch_ref[...]
        out_ref[:, :L, :] = jnp.transpose(x, (1, 0, 2))
        token = speed_ref[...] * fcw_ref[...] + fcb_ref[...]
        out_ref[:, L, :] = token

    @pl.when(t == 1)
    def _temb_plane():
        col = temb_ref[:, 0:1]
        row = jnp.transpose(col, (1, 0))
        out_ref[0, :, :] = jnp.broadcast_to(row, (L + 1, BC))


@jax.jit
def kernel(patch, speed, time_step, fc_w, fc_b, time_table):
    patch_t = jnp.transpose(patch, (1, 2, 0))
    speed_row = speed.reshape(1, B)
    fcw_col = fc_w.reshape(D, 1)
    fcb_col = fc_b.reshape(D, 1)
    table_rows = jnp.pad(time_table, ((0, TBL - (MAX_T + 1)), (0, 127)))
    ts_1d = time_step.astype(jnp.int32)

    temb_rows = _sc_gather(ts_1d, table_rows)

    out_t = pl.pallas_call(
        _body,
        grid=(2, NB),
        in_specs=[
            pl.BlockSpec((BC, 128), lambda t, i: (i * t, 0)),
            pl.BlockSpec((1, BC), lambda t, i: (0, i * (1 - t))),
            pl.BlockSpec((D, 1), lambda t, i: (0, 0)),
            pl.BlockSpec((D, 1), lambda t, i: (0, 0)),
            pl.BlockSpec((L, D, BC), lambda t, i: (0, 0, i * (1 - t))),
        ],
        out_specs=pl.BlockSpec((D, L + 1, BC), lambda t, i: (t, 0, i)),
        out_shape=jax.ShapeDtypeStruct((D + 1, L + 1, B), jnp.float32),
    )(temb_rows, speed_row, fcw_col, fcb_col, patch_t)
    return jnp.transpose(out_t, (2, 1, 0))


# speedup vs baseline: 1.0985x; 1.0985x over previous
import functools

import jax
import jax.numpy as jnp
from jax import lax
from jax.experimental import pallas as pl
from jax.experimental.pallas import tpu as pltpu
from jax.experimental.pallas import tpu_sc as plsc

B = 4096
L = 196
D = 64
MAX_T = 160
TBL = 168
BC = 128

SC_NC = 2
SC_NW = 32
BPW = B // SC_NW


def _sc_gather_body(ts_hbm, table_hbm, out_hbm, idx_v, rows_v, sem):
    wid = lax.axis_index("s") * SC_NC + lax.axis_index("c")
    base = wid * BPW
    pltpu.sync_copy(ts_hbm.at[pl.ds(base, BPW)], idx_v)
    pltpu.async_copy(table_hbm.at[idx_v], rows_v, sem).wait()
    pltpu.sync_copy(rows_v, out_hbm.at[pl.ds(base, BPW)])


_sc_gather = functools.partial(
    pl.kernel,
    out_type=jax.ShapeDtypeStruct((B, 128), jnp.float32),
    mesh=plsc.VectorSubcoreMesh(core_axis_name="c", subcore_axis_name="s"),
    scratch_types=[
        pltpu.VMEM((BPW,), jnp.int32),
        pltpu.VMEM((BPW, 128), jnp.float32),
        pltpu.SemaphoreType.DMA,
    ],
)(_sc_gather_body)


def _body(speed_ref, fcw_ref, fcb_ref, patch_ref, out_ref):
    x = patch_ref[...]
    out_ref[:, :L, :] = jnp.transpose(x, (1, 0, 2))
    token = speed_ref[...] * fcw_ref[...] + fcb_ref[...]
    out_ref[:, L, :] = token


def _finish_body(temb_ref, big_ref, out_ref):
    del big_ref
    col = temb_ref[:, 0:1]
    row = jnp.transpose(col, (1, 0))
    out_ref[0, :, :] = jnp.broadcast_to(row, (L + 1, B))


@jax.jit
def kernel(patch, speed, time_step, fc_w, fc_b, time_table):
    patch_t = jnp.transpose(patch, (1, 2, 0))
    speed_row = speed.reshape(1, B)
    fcw_col = fc_w.reshape(D, 1)
    fcb_col = fc_b.reshape(D, 1)
    table_rows = jnp.pad(time_table, ((0, TBL - (MAX_T + 1)), (0, 127)))
    ts_1d = time_step.astype(jnp.int32)

    temb_rows = _sc_gather(ts_1d, table_rows)

    grid = (B // BC,)
    big = pl.pallas_call(
        _body,
        grid=grid,
        in_specs=[
            pl.BlockSpec((1, BC), lambda i: (0, i)),
            pl.BlockSpec((D, 1), lambda i: (0, 0)),
            pl.BlockSpec((D, 1), lambda i: (0, 0)),
            pl.BlockSpec((L, D, BC), lambda i: (0, 0, i)),
        ],
        out_specs=pl.BlockSpec((D, L + 1, BC), lambda i: (0, 0, i)),
        out_shape=jax.ShapeDtypeStruct((D + 1, L + 1, B), jnp.float32),
    )(speed_row, fcw_col, fcb_col, patch_t)

    out_t = pl.pallas_call(
        _finish_body,
        grid=(1,),
        in_specs=[
            pl.BlockSpec((B, 128), lambda i: (0, 0)),
            pl.BlockSpec(memory_space=pltpu.MemorySpace.HBM),
        ],
        out_specs=pl.BlockSpec((1, L + 1, B), lambda i: (D, 0, 0)),
        out_shape=jax.ShapeDtypeStruct((D + 1, L + 1, B), jnp.float32),
        input_output_aliases={1: 0},
    )(temb_rows, big)
    return jnp.transpose(out_t, (2, 1, 0))
